# baseline (speedup 1.0000x reference)
import jax
import jax.numpy as jnp
from jax.experimental import pallas as pl
from jax.experimental.pallas import tpu as pltpu

_B, _N, _D, _P = 32, 64, 512, 72
_MAX_EDGES = 172
_G = 32


def _dot(a, b, precision=jax.lax.Precision.HIGHEST):
    return jnp.dot(a, b, preferred_element_type=jnp.float32,
                   precision=precision)


def _dgen(a, b, dims, precision=jax.lax.Precision.HIGHEST):
    return jax.lax.dot_general(a, b, dims, preferred_element_type=jnp.float32,
                               precision=precision)


def _body(num_obj_ref, num_edges_ref,
          x_ref, pairs_ref,
          sW1, sb1, sW2, sb2, oW1, ob1, oW2, ob2,
          gW1, gb1, gW2, gb2,
          cW1, cb1, cW2, cb2,
          lW1, lb1, lW2, lb2,
          mW1, mb1, mW2, mb2,
          lr_ref, cr_ref, mr_ref):
    step = pl.program_id(0)
    dflt = jax.lax.Precision.DEFAULT
    relu = lambda v: jnp.maximum(v, 0.0)

    x = x_ref[...]
    xs = x.reshape(_G * _N, _D)

    phi_all = (_dot(relu(_dot(xs, sW1[...], dflt) + sb1[...]), sW2[...], dflt)
               + sb2[...])
    psi_all = (_dot(relu(_dot(xs, oW1[...], dflt) + ob1[...]), oW2[...], dflt)
               + ob2[...])

    row_i = jax.lax.broadcasted_iota(jnp.int32, (_N, _N), 0)
    col_j = jax.lax.broadcasted_iota(jnp.int32, (_N, _N), 1)
    lane = jax.lax.broadcasted_iota(jnp.int32, (_P, _N), 1)
    prow = jax.lax.broadcasted_iota(jnp.int32, (_P, 1), 0)
    upper_inc = (row_i <= col_j).astype(jnp.float32)
    lower_strict = (row_i > col_j).astype(jnp.float32)

    nobjs = [num_obj_ref[step * _G + g] for g in range(_G)]
    nedges = [num_edges_ref[step * _G + g] for g in range(_G)]

    slogs = [
        _dgen(phi_all[g * _N:(g + 1) * _N], psi_all[g * _N:(g + 1) * _N],
              (((1,), (1,)), ((), ())), dflt)
        for g in range(_G)
    ]

    Ss = [(lane == pairs_ref[g][:, 0:1]).astype(jnp.float32) for g in range(_G)]
    Dhs = [(lane == pairs_ref[g][:, 1:2]).astype(jnp.float32) for g in range(_G)]
    pms = [(prow < nedges[g]).astype(jnp.float32) for g in range(_G)]
    extras = [
        (_dgen(Ss[g], Dhs[g] * pms[g], (((0,), (0,)), ((), ())))
         + _dgen(Dhs[g], Ss[g] * pms[g], (((0,), (0,)), ((), ()))))
        for g in range(_G)
    ]
    rels = [
        ((slogs[g] > 0.0) & (row_i < nobjs[g]) & (col_j < nobjs[g]))
        | (extras[g] > 0.0)
        for g in range(_G)
    ]
    relfs = [jnp.where(rels[g], 1.0, 0.0) for g in range(_G)]

    withins = [_dot(relfs[g], upper_inc) for g in range(_G)]
    prefixes = [
        _dot(lower_strict, withins[g][:, _N - 1:_N]) for g in range(_G)
    ]
    ws = [
        jnp.where(rels[g] & (withins[g] + prefixes[g] <= float(_MAX_EDGES)),
                  1.0, 0.0)
        for g in range(_G)
    ]

    a_parts = [_dgen(ws[g], x[g], (((0,), (0,)), ((), ()))) for g in range(_G)]
    a_all = jnp.concatenate(a_parts, axis=0)
    h_all = relu(_dot(a_all, gW1[...], dflt) + gb1[...])

    a2_parts = [
        _dgen(ws[g], h_all[g * _N:(g + 1) * _N], (((0,), (0,)), ((), ())))
        for g in range(_G)
    ]
    a2_all = jnp.concatenate(a2_parts, axis=0)
    h2_all = _dot(a2_all, gW2[...], dflt) + gb2[...]

    valid_col = (jax.lax.broadcasted_iota(jnp.int32, (_N, 1), 0))
    ci_parts = []
    for g in range(_G):
        emb = jnp.where(valid_col < nobjs[g],
                        h2_all[g * _N:(g + 1) * _N], 0.0)
        ci_parts.append(_dot(Ss[g], emb) + _dot(Dhs[g], emb))
    ci = jnp.concatenate(ci_parts, axis=0)

    def head(W1, b1, W2, b2):
        return (_dot(relu(_dot(ci, W1[...], dflt) + b1[...]), W2[...], dflt)
                + b2[...])

    lr_ref[...] = head(lW1, lb1, lW2, lb2).reshape(_G, _P, 6)
    crl = head(cW1, cb1, cW2, cb2)
    crl = crl - jnp.max(crl, axis=1, keepdims=True)
    e = jnp.exp(crl)
    cr_ref[...] = (e / jnp.sum(e, axis=1, keepdims=True)).reshape(_G, _P, 3)
    mr_ref[...] = head(mW1, mb1, mW2, mb2).reshape(_G, _P, 3)


def kernel(concatenated_node_features, num_obj, num_edges, object_pairs,
           sub_W1, sub_b1, sub_W2, sub_b2, obj_W1, obj_b1, obj_W2, obj_b2,
           gnn_W1, gnn_b1, gnn_W2, gnn_b2, cr_W1, cr_b1, cr_W2, cr_b2,
           lr_W1, lr_b1, lr_W2, lr_b2, mr_W1, mr_b1, mr_W2, mr_b2):
    x = concatenated_node_features
    pairs = object_pairs

    weights = [sub_W1, sub_b1, sub_W2, sub_b2,
               obj_W1, obj_b1, obj_W2, obj_b2,
               gnn_W1, gnn_b1, gnn_W2, gnn_b2,
               cr_W1, cr_b1, cr_W2, cr_b2,
               lr_W1, lr_b1, lr_W2, lr_b2,
               mr_W1, mr_b1, mr_W2, mr_b2]

    def rep_spec(a):
        return pl.BlockSpec(a.shape, lambda s, *_: (0,) * a.ndim)

    grid_spec = pltpu.PrefetchScalarGridSpec(
        num_scalar_prefetch=2,
        grid=(_B // _G,),
        in_specs=[
            pl.BlockSpec((_G, _N, _D), lambda s, *_: (s, 0, 0)),
            pl.BlockSpec((_G, _P, 2), lambda s, *_: (s, 0, 0)),
        ] + [rep_spec(a) for a in weights],
        out_specs=[
            pl.BlockSpec((_G, _P, 6), lambda s, *_: (s, 0, 0)),
            pl.BlockSpec((_G, _P, 3), lambda s, *_: (s, 0, 0)),
            pl.BlockSpec((_G, _P, 3), lambda s, *_: (s, 0, 0)),
        ],
    )

    lr, cr, mr = pl.pallas_call(
        _body,
        grid_spec=grid_spec,
        out_shape=[
            jax.ShapeDtypeStruct((_B, _P, 6), jnp.float32),
            jax.ShapeDtypeStruct((_B, _P, 3), jnp.float32),
            jax.ShapeDtypeStruct((_B, _P, 3), jnp.float32),
        ],
    )(num_obj, num_edges, x, pairs, *weights)
    return (lr, cr, mr)


# speedup vs baseline: 1.3728x; 1.3728x over previous
import jax
import jax.numpy as jnp
from jax.experimental import pallas as pl
from jax.experimental.pallas import tpu as pltpu

_B, _N, _D, _P = 32, 64, 512, 72
_MAX_EDGES = 172
_G = 16


def _dot(a, b, precision=jax.lax.Precision.HIGHEST):
    return jnp.dot(a, b, preferred_element_type=jnp.float32,
                   precision=precision)


def _dgen(a, b, dims, precision=jax.lax.Precision.HIGHEST):
    return jax.lax.dot_general(a, b, dims, preferred_element_type=jnp.float32,
                               precision=precision)


def _body(num_obj_ref, num_edges_ref,
          x_ref, pairs_ref,
          sW1, sb1, sW2, sb2, oW1, ob1, oW2, ob2,
          gW1, gb1, gW2, gb2,
          cW1, cb1, cW2, cb2,
          lW1, lb1, lW2, lb2,
          mW1, mb1, mW2, mb2,
          lr_ref, cr_ref, mr_ref):
    step = pl.program_id(0)
    dflt = jax.lax.Precision.DEFAULT
    relu = lambda v: jnp.maximum(v, 0.0)

    x = x_ref[...]
    xs = x.reshape(_G * _N, _D)

    phi_all = (_dot(relu(_dot(xs, sW1[...], dflt) + sb1[...]), sW2[...], dflt)
               + sb2[...])
    psi_all = (_dot(relu(_dot(xs, oW1[...], dflt) + ob1[...]), oW2[...], dflt)
               + ob2[...])

    row_i = jax.lax.broadcasted_iota(jnp.int32, (_N, _N), 0)
    col_j = jax.lax.broadcasted_iota(jnp.int32, (_N, _N), 1)
    lane = jax.lax.broadcasted_iota(jnp.int32, (_P, _N), 1)
    prow = jax.lax.broadcasted_iota(jnp.int32, (_P, 1), 0)
    upper_inc = (row_i <= col_j).astype(jnp.float32)
    lower_strict = (row_i > col_j).astype(jnp.float32)

    nobjs = [num_obj_ref[step * _G + g] for g in range(_G)]
    nedges = [num_edges_ref[step * _G + g] for g in range(_G)]

    slogs = [
        _dgen(phi_all[g * _N:(g + 1) * _N], psi_all[g * _N:(g + 1) * _N],
              (((1,), (1,)), ((), ())), dflt)
        for g in range(_G)
    ]

    Ss = [(lane == pairs_ref[g][:, 0:1]).astype(jnp.float32) for g in range(_G)]
    Dhs = [(lane == pairs_ref[g][:, 1:2]).astype(jnp.float32) for g in range(_G)]
    pms = [(prow < nedges[g]).astype(jnp.float32) for g in range(_G)]
    extras = [
        (_dgen(Ss[g], Dhs[g] * pms[g], (((0,), (0,)), ((), ())), dflt)
         + _dgen(Dhs[g], Ss[g] * pms[g], (((0,), (0,)), ((), ())), dflt))
        for g in range(_G)
    ]
    rels = [
        ((slogs[g] > 0.0) & (row_i < nobjs[g]) & (col_j < nobjs[g]))
        | (extras[g] > 0.0)
        for g in range(_G)
    ]
    relfs = [jnp.where(rels[g], 1.0, 0.0) for g in range(_G)]

    withins = [_dot(relfs[g], upper_inc, dflt) for g in range(_G)]
    prefixes = [
        _dot(lower_strict, withins[g][:, _N - 1:_N], dflt)
        for g in range(_G)
    ]
    ws = [
        jnp.where(rels[g] & (withins[g] + prefixes[g] <= float(_MAX_EDGES)),
                  1.0, 0.0)
        for g in range(_G)
    ]

    def _hilo(v):
        hi = v.astype(jnp.bfloat16).astype(jnp.float32)
        return hi, v - hi

    prop2 = (((0,), (0,)), ((), ()))

    xs_hi, xs_lo = _hilo(xs)
    a_parts = [
        _dgen(ws[g], xs_hi[g * _N:(g + 1) * _N], prop2, dflt)
        + _dgen(ws[g], xs_lo[g * _N:(g + 1) * _N], prop2, dflt)
        for g in range(_G)
    ]
    a_all = jnp.concatenate(a_parts, axis=0)
    h_all = relu(_dot(a_all, gW1[...], dflt) + gb1[...])

    h_hi, h_lo = _hilo(h_all)
    a2_parts = [
        _dgen(ws[g], h_hi[g * _N:(g + 1) * _N], prop2, dflt)
        + _dgen(ws[g], h_lo[g * _N:(g + 1) * _N], prop2, dflt)
        for g in range(_G)
    ]
    a2_all = jnp.concatenate(a2_parts, axis=0)
    h2_all = _dot(a2_all, gW2[...], dflt) + gb2[...]

    valid_col = (jax.lax.broadcasted_iota(jnp.int32, (_N, 1), 0))
    ci_parts = []
    for g in range(_G):
        emb = jnp.where(valid_col < nobjs[g],
                        h2_all[g * _N:(g + 1) * _N], 0.0)
        e_hi, e_lo = _hilo(emb)
        sd = Ss[g] + Dhs[g]
        ci_parts.append(_dot(sd, e_hi, dflt) + _dot(sd, e_lo, dflt))
    ci = jnp.concatenate(ci_parts, axis=0)

    def head(W1, b1, W2, b2):
        return (_dot(relu(_dot(ci, W1[...], dflt) + b1[...]), W2[...], dflt)
                + b2[...])

    lr_ref[...] = head(lW1, lb1, lW2, lb2).reshape(_G, _P, 6)
    crl = head(cW1, cb1, cW2, cb2)
    crl = crl - jnp.max(crl, axis=1, keepdims=True)
    e = jnp.exp(crl)
    cr_ref[...] = (e / jnp.sum(e, axis=1, keepdims=True)).reshape(_G, _P, 3)
    mr_ref[...] = head(mW1, mb1, mW2, mb2).reshape(_G, _P, 3)


def kernel(concatenated_node_features, num_obj, num_edges, object_pairs,
           sub_W1, sub_b1, sub_W2, sub_b2, obj_W1, obj_b1, obj_W2, obj_b2,
           gnn_W1, gnn_b1, gnn_W2, gnn_b2, cr_W1, cr_b1, cr_W2, cr_b2,
           lr_W1, lr_b1, lr_W2, lr_b2, mr_W1, mr_b1, mr_W2, mr_b2):
    x = concatenated_node_features
    pairs = object_pairs

    weights = [sub_W1, sub_b1, sub_W2, sub_b2,
               obj_W1, obj_b1, obj_W2, obj_b2,
               gnn_W1, gnn_b1, gnn_W2, gnn_b2,
               cr_W1, cr_b1, cr_W2, cr_b2,
               lr_W1, lr_b1, lr_W2, lr_b2,
               mr_W1, mr_b1, mr_W2, mr_b2]

    def rep_spec(a):
        return pl.BlockSpec(a.shape, lambda s, *_: (0,) * a.ndim)

    grid_spec = pltpu.PrefetchScalarGridSpec(
        num_scalar_prefetch=2,
        grid=(_B // _G,),
        in_specs=[
            pl.BlockSpec((_G, _N, _D), lambda s, *_: (s, 0, 0)),
            pl.BlockSpec((_G, _P, 2), lambda s, *_: (s, 0, 0)),
        ] + [rep_spec(a) for a in weights],
        out_specs=[
            pl.BlockSpec((_G, _P, 6), lambda s, *_: (s, 0, 0)),
            pl.BlockSpec((_G, _P, 3), lambda s, *_: (s, 0, 0)),
            pl.BlockSpec((_G, _P, 3), lambda s, *_: (s, 0, 0)),
        ],
    )

    lr, cr, mr = pl.pallas_call(
        _body,
        grid_spec=grid_spec,
        out_shape=[
            jax.ShapeDtypeStruct((_B, _P, 6), jnp.float32),
            jax.ShapeDtypeStruct((_B, _P, 3), jnp.float32),
            jax.ShapeDtypeStruct((_B, _P, 3), jnp.float32),
        ],
    )(num_obj, num_edges, x, pairs, *weights)
    return (lr, cr, mr)
